# baseline (speedup 1.0000x reference)
import functools

import jax
import jax.numpy as jnp
import numpy as np
from jax import lax
from jax.experimental import pallas as pl
from jax.experimental.pallas import tpu as pltpu
from jax.experimental.pallas import tpu_sc as plsc

N = 10000
E = 320000
F = 128

NC = 2
NS = 16
NW = NC * NS
CHUNK = 128
NCHUNKS = E // CHUNK
BASE_CH = NCHUNKS // NW
REM_CH = NCHUNKS - BASE_CH * NW

_SC_MESH = dict(
    mesh=plsc.VectorSubcoreMesh(core_axis_name="c", subcore_axis_name="s"),
    compiler_params=pltpu.CompilerParams(needs_layout_passes=False),
)


NT = 80


def _worker_ids():
    cid = lax.axis_index("c")
    sid = lax.axis_index("s")
    wid = sid * NC + cid
    return cid, sid, wid


def _hist_accum(hist_ref, idx_ref, val16):
    for l in range(CHUNK // 16):
        idx16 = idx_ref[pl.ds(l * 16, 16)]
        plsc.addupdate_scatter(hist_ref, [idx16], val16)


def _k1_body(x_hbm, idxi_hbm, idxj_hbm, zeros2d_hbm, zeros1d_hbm,
             gj_hbm, ssum_hbm, hist_hbm,
             idxi0, idxi1, idxj0, idxj1, rj0, rj1,
             hist_v, ssum_sh, sj0, sj1):
    cid, sid, wid = _worker_ids()
    idxi = (idxi0, idxi1)
    idxj = (idxj0, idxj1)
    rj = (rj0, rj1)
    sj = (sj0, sj1)

    @pl.when(sid == 0)
    def _():
        pltpu.sync_copy(zeros2d_hbm, ssum_sh)

    pltpu.sync_copy(zeros1d_hbm, hist_v)
    plsc.subcore_barrier()

    ones16 = jnp.ones((16,), jnp.float32)

    def load_and_fire(t, b):
        c = wid + NW * t

        @pl.when(c < NCHUNKS)
        def _():
            base = c * CHUNK
            pltpu.sync_copy(idxi_hbm.at[pl.ds(base, CHUNK)], idxi[b])
            pltpu.sync_copy(idxj_hbm.at[pl.ds(base, CHUNK)], idxj[b])
            pltpu.async_copy(x_hbm.at[idxj[b]], rj[b], sj[b])

    for b in range(2):
        load_and_fire(jnp.int32(b), b)

    @pl.loop(0, NT, step=2)
    def _(t0):
        for b in range(2):
            t = t0 + b
            c = wid + NW * t

            @pl.when(c < NCHUNKS)
            def _():
                base = c * CHUNK
                pltpu.make_async_copy(x_hbm.at[idxj[b]], rj[b], sj[b]).wait()
                pltpu.sync_copy(rj[b], gj_hbm.at[pl.ds(base, CHUNK)])
                pltpu.sync_copy(rj[b], ssum_sh.at[idxi[b]], add=True)
                _hist_accum(hist_v, idxi[b], ones16)

            load_and_fire(t + 2, b)

    pltpu.sync_copy(hist_v, hist_hbm.at[wid])
    plsc.subcore_barrier()

    @pl.when(sid == 0)
    def _():
        pltpu.sync_copy(ssum_sh, ssum_hbm.at[cid])


def _k2_body(u_hbm, idxi_hbm, gib_hbm,
             idxi0, idxi1, r0, r1, s0, s1):
    _, _, wid = _worker_ids()
    idxi = (idxi0, idxi1)
    rows = (r0, r1)
    sem = (s0, s1)

    def load_and_fire(t, b):
        c = wid + NW * t

        @pl.when(c < NCHUNKS)
        def _():
            base = c * CHUNK
            pltpu.sync_copy(idxi_hbm.at[pl.ds(base, CHUNK)], idxi[b])
            pltpu.async_copy(u_hbm.at[idxi[b]], rows[b], sem[b])

    for b in range(2):
        load_and_fire(jnp.int32(b), b)

    @pl.loop(0, NT, step=2)
    def _(t0):
        for b in range(2):
            t = t0 + b
            c = wid + NW * t

            @pl.when(c < NCHUNKS)
            def _():
                base = c * CHUNK
                pltpu.make_async_copy(u_hbm.at[idxi[b]], rows[b], sem[b]).wait()
                pltpu.sync_copy(rows[b], gib_hbm.at[pl.ds(base, CHUNK)])

            load_and_fire(t + 2, b)


def _k3_body(w_hbm, ex_hbm, idxi_hbm, zeros2d_hbm, zeros1d_hbm,
             wsum_hbm, dhist_hbm,
             idxi0, idxi1, r0, r1, ex0, ex1, dhist_v, wsum_sh, s0, s1):
    cid, sid, wid = _worker_ids()
    idxi = (idxi0, idxi1)
    rows = (r0, r1)
    exv = (ex0, ex1)
    sem = (s0, s1)

    @pl.when(sid == 0)
    def _():
        pltpu.sync_copy(zeros2d_hbm, wsum_sh)

    pltpu.sync_copy(zeros1d_hbm, dhist_v)
    plsc.subcore_barrier()

    def load_and_fire(t, b):
        c = wid + NW * t

        @pl.when(c < NCHUNKS)
        def _():
            base = c * CHUNK
            pltpu.sync_copy(idxi_hbm.at[pl.ds(base, CHUNK)], idxi[b])
            pltpu.sync_copy(ex_hbm.at[pl.ds(base, CHUNK)], exv[b])
            pltpu.async_copy(w_hbm.at[pl.ds(base, CHUNK)], rows[b], sem[b])

    for b in range(2):
        load_and_fire(jnp.int32(b), b)

    @pl.loop(0, NT, step=2)
    def _(t0):
        for b in range(2):
            t = t0 + b
            c = wid + NW * t

            @pl.when(c < NCHUNKS)
            def _():
                pltpu.make_async_copy(
                    w_hbm.at[pl.ds(c * CHUNK, CHUNK)], rows[b], sem[b]).wait()
                pltpu.sync_copy(rows[b], wsum_sh.at[idxi[b]], add=True)
                for l in range(CHUNK // 16):
                    idx16 = idxi[b][pl.ds(l * 16, 16)]
                    ex16 = exv[b][pl.ds(l * 16, 16)]
                    plsc.addupdate_scatter(dhist_v, [idx16], ex16)

            load_and_fire(t + 2, b)

    pltpu.sync_copy(dhist_v, dhist_hbm.at[wid])
    plsc.subcore_barrier()

    @pl.when(sid == 0)
    def _():
        pltpu.sync_copy(wsum_sh, wsum_hbm.at[cid])


_f32 = jnp.float32


def _sc_gather_scatter_pass1(x, idx_i, idx_j):
    zeros2d = jnp.zeros((N, F), _f32)
    zeros1d = jnp.zeros((N,), _f32)
    k1 = pl.kernel(
        _k1_body,
        out_type=(
            jax.ShapeDtypeStruct((E, F), _f32),
            jax.ShapeDtypeStruct((NC, N, F), _f32),
            jax.ShapeDtypeStruct((NW, N), _f32),
        ),
        scratch_types=(
            [pltpu.VMEM((CHUNK,), jnp.int32)] * 4
            + [pltpu.VMEM((CHUNK, F), _f32)] * 2
            + [pltpu.VMEM((N,), _f32), pltpu.VMEM_SHARED((N, F), _f32)]
            + [pltpu.SemaphoreType.DMA] * 2
        ),
        **_SC_MESH,
    )
    return k1(x, idx_i, idx_j, zeros2d, zeros1d)


def _sc_gather_xi_b(u_tab, idx_i):
    k2 = pl.kernel(
        _k2_body,
        out_type=jax.ShapeDtypeStruct((E, F), jnp.int32),
        scratch_types=(
            [pltpu.VMEM((CHUNK,), jnp.int32)] * 2
            + [pltpu.VMEM((CHUNK, F), jnp.int32)] * 2
            + [pltpu.SemaphoreType.DMA] * 2
        ),
        **_SC_MESH,
    )
    return k2(u_tab, idx_i)


def _sc_scatter_pass3(weighted, e_x, idx_i):
    zeros2d = jnp.zeros((N, F), _f32)
    zeros1d = jnp.zeros((N,), _f32)
    k3 = pl.kernel(
        _k3_body,
        out_type=(
            jax.ShapeDtypeStruct((NC, N, F), _f32),
            jax.ShapeDtypeStruct((NW, N), _f32),
        ),
        scratch_types=(
            [pltpu.VMEM((CHUNK,), jnp.int32)] * 2
            + [pltpu.VMEM((CHUNK, F), _f32)] * 2
            + [pltpu.VMEM((CHUNK,), _f32)] * 2
            + [pltpu.VMEM((N,), _f32), pltpu.VMEM_SHARED((N, F), _f32)]
            + [pltpu.SemaphoreType.DMA] * 2
        ),
        **_SC_MESH,
    )
    return k3(weighted, e_x, idx_i, zeros2d, zeros1d)


_TROWS = 1000


def _t1_body(x_ref, ssum_ref, histt_ref, w1b_ref, b_ref, counts_ref):
    counts = jnp.sum(histt_ref[...], axis=1, keepdims=True)
    cmax = jnp.maximum(counts, 1.0)
    ga = (counts * x_ref[...] + ssum_ref[0] + ssum_ref[1]) * 0.5 / cmax
    b_ref[...] = jnp.dot(ga, w1b_ref[...], preferred_element_type=_f32)
    counts_ref[...] = counts


def _tc_combine(x, ssum_p, hist, w1b):
    histt = hist.T
    grid = (N // _TROWS,)
    return pl.pallas_call(
        _t1_body,
        grid=grid,
        in_specs=[
            pl.BlockSpec((_TROWS, F), lambda b: (b, 0)),
            pl.BlockSpec((NC, _TROWS, F), lambda b: (0, b, 0)),
            pl.BlockSpec((_TROWS, NW), lambda b: (b, 0)),
            pl.BlockSpec((F, F), lambda b: (0, 0)),
        ],
        out_specs=[
            pl.BlockSpec((_TROWS, F), lambda b: (b, 0)),
            pl.BlockSpec((_TROWS, 1), lambda b: (b, 0)),
        ],
        out_shape=[
            jax.ShapeDtypeStruct((N, F), _f32),
            jax.ShapeDtypeStruct((N, 1), _f32),
        ],
    )(x, ssum_p, histt, w1b)


_EBLK = 1280
_BF = jnp.bfloat16


def _t2_body(gib_ref, gj_ref,
             we_ref, wo_ref, wf_ref, eb1_ref, gb1_ref,
             w2m_ref, b2m_ref, w3m_ref, eb3_ref, gb3_ref,
             se_ref, so_ref, sw1b_ref, sw1c_ref, sb1_ref,
             sw2_ref, sb2_ref, sw3_ref, sb3_ref,
             w_ref, ex_ref):
    gib = gib_ref[...]
    ev16 = lax.bitcast_convert_type(
        jnp.left_shift(gib, 16), _f32).astype(_BF)
    od16 = lax.bitcast_convert_type(
        jnp.bitwise_and(gib, jnp.int32(-65536)), _f32).astype(_BF)
    gj16 = gj_ref[...].astype(_BF)

    def mm(a, b):
        return jnp.dot(a, b, preferred_element_type=_f32)

    t = (mm(ev16, we_ref[...]) + mm(od16, wo_ref[...])
         + mm(gj16, wf_ref[...]))
    h = jnp.maximum(t[:, :128] + eb1_ref[...], 0.0)
    g = jnp.maximum(t[:, 128:] + gb1_ref[...], 0.0)
    hg = jnp.concatenate([h, g], axis=1).astype(_BF)
    u = jnp.maximum(mm(hg, w2m_ref[...]) + b2m_ref[...], 0.0)
    v = mm(u.astype(_BF), w3m_ref[...])
    eij = v[:, 0:1] + eb3_ref[...]
    edge = v[:, 1:2] + gb3_ref[...]
    e_x = jnp.exp(eij)

    s = jnp.maximum(mm(ev16, se_ref[...]) + mm(od16, so_ref[...])
                    + mm(gj16, sw1b_ref[...])
                    + edge * sw1c_ref[...] + sb1_ref[...], 0.0)
    s = jnp.maximum(mm(s.astype(_BF), sw2_ref[...]) + sb2_ref[...], 0.0)
    m = mm(s.astype(_BF), sw3_ref[...]) + sb3_ref[...]

    w_ref[...] = e_x * m
    ex_ref[...] = e_x


def _tc_mlp(gib, gj, weights):
    grid = (E // _EBLK,)
    eblk = lambda b: (b, 0)
    full = lambda b: (0, 0)
    wspecs = [pl.BlockSpec(w.shape, full) for w in weights]
    return pl.pallas_call(
        _t2_body,
        grid=grid,
        in_specs=[
            pl.BlockSpec((_EBLK, F), eblk),
            pl.BlockSpec((_EBLK, F), eblk),
        ] + wspecs,
        out_specs=[
            pl.BlockSpec((_EBLK, F), eblk),
            pl.BlockSpec((_EBLK, 1), eblk),
        ],
        out_shape=[
            jax.ShapeDtypeStruct((E, F), _f32),
            jax.ShapeDtypeStruct((E, 1), _f32),
        ],
    )(gib, gj, *weights)


def _t3_body(x_ref, wsum_ref, dhistt_ref, counts_ref, out_ref):
    dsum = jnp.sum(dhistt_ref[...], axis=1, keepdims=True)
    cmax = jnp.maximum(counts_ref[...], 1.0)
    agg = (wsum_ref[0] + wsum_ref[1]) / ((dsum + 1e-9) * cmax)
    out_ref[...] = jnp.maximum(x_ref[...] + agg, 0.0)


def _tc_final(x, wsum_p, dhist, counts):
    dhistt = dhist.T
    grid = (N // _TROWS,)
    return pl.pallas_call(
        _t3_body,
        grid=grid,
        in_specs=[
            pl.BlockSpec((_TROWS, F), lambda b: (b, 0)),
            pl.BlockSpec((NC, _TROWS, F), lambda b: (0, b, 0)),
            pl.BlockSpec((_TROWS, NW), lambda b: (b, 0)),
            pl.BlockSpec((_TROWS, 1), lambda b: (b, 0)),
        ],
        out_specs=pl.BlockSpec((_TROWS, F), lambda b: (b, 0)),
        out_shape=jax.ShapeDtypeStruct((N, F), _f32),
    )(x, wsum_p, dhistt, counts)


def kernel(x, edge_index, e,
           enet_W1, enet_b1, enet_W2, enet_b2, enet_W3, enet_b3,
           edgenet_W1, edgenet_b1, edgenet_W2, edgenet_b2, edgenet_W3, edgenet_b3,
           snet_W1, snet_b1, snet_W2, snet_b2, snet_W3, snet_b3):
    idx_i = edge_index[0]
    idx_j = edge_index[1]

    gj, ssum_p, hist = _sc_gather_scatter_pass1(x, idx_i, idx_j)

    b_tab, counts = _tc_combine(x, ssum_p, hist, edgenet_W1[F:])

    def pack_rows(a):
        return lax.bitcast_convert_type(
            a.astype(_BF).reshape(N, F // 2, 2), jnp.int32)

    u_tab = jnp.concatenate([pack_rows(x), pack_rows(b_tab)], axis=1)
    gib = _sc_gather_xi_b(u_tab, idx_i)

    w1mh = jnp.concatenate([enet_W1, edgenet_W1[:F]],
                           axis=1) * 0.5
    H = F // 2
    ide = np.zeros((H, 2 * F), np.float32)
    ide[np.arange(H), F + 2 * np.arange(H)] = 1.0
    ido = np.zeros((H, 2 * F), np.float32)
    ido[np.arange(H), F + 2 * np.arange(H) + 1] = 1.0
    we = jnp.concatenate([w1mh[0::2], jnp.asarray(ide)], axis=0).astype(_BF)
    wo = jnp.concatenate([w1mh[1::2], jnp.asarray(ido)], axis=0).astype(_BF)
    z128_64 = jnp.zeros((F, 64), _f32)
    w2m = jnp.concatenate([
        jnp.concatenate([enet_W2, z128_64], axis=1),
        jnp.concatenate([z128_64, edgenet_W2], axis=1),
    ], axis=0).astype(_BF)
    b2m = jnp.concatenate([enet_b2, edgenet_b2]).reshape(1, -1)
    z64_1 = jnp.zeros((64, 1), _f32)
    w3m = jnp.concatenate([
        jnp.concatenate([enet_W3, z64_1], axis=1),
        jnp.concatenate([z64_1, edgenet_W3], axis=1),
    ], axis=0).astype(_BF)
    sw1a = snet_W1[:F]
    zh = jnp.zeros((H, F), _f32)
    se = jnp.concatenate([sw1a[0::2], zh], axis=0).astype(_BF)
    so = jnp.concatenate([sw1a[1::2], zh], axis=0).astype(_BF)
    weights = (
        we, wo, w1mh.astype(_BF),
        enet_b1.reshape(1, -1), edgenet_b1.reshape(1, -1),
        w2m, b2m, w3m,
        enet_b3.reshape(1, -1), edgenet_b3.reshape(1, -1),
        se, so,
        snet_W1[F:2 * F].astype(_BF),
        snet_W1[2 * F:], snet_b1.reshape(1, -1),
        snet_W2.astype(_BF), snet_b2.reshape(1, -1),
        snet_W3.astype(_BF), snet_b3.reshape(1, -1),
    )
    weighted, e_x = _tc_mlp(gib, gj, weights)

    wsum_p, dhist = _sc_scatter_pass3(weighted, e_x[:, 0], idx_i)

    return _tc_final(x, wsum_p, dhist, counts)


# speedup vs baseline: 1.0929x; 1.0929x over previous
import functools

import jax
import jax.numpy as jnp
import numpy as np
from jax import lax
from jax.experimental import pallas as pl
from jax.experimental.pallas import tpu as pltpu
from jax.experimental.pallas import tpu_sc as plsc

N = 10000
E = 320000
F = 128

NC = 2
NS = 16
NW = NC * NS
CHUNK = 128
NCHUNKS = E // CHUNK
BASE_CH = NCHUNKS // NW
REM_CH = NCHUNKS - BASE_CH * NW

_SC_MESH = dict(
    mesh=plsc.VectorSubcoreMesh(core_axis_name="c", subcore_axis_name="s"),
    compiler_params=pltpu.CompilerParams(needs_layout_passes=False),
)


NT = 80


def _worker_ids():
    cid = lax.axis_index("c")
    sid = lax.axis_index("s")
    wid = sid * NC + cid
    return cid, sid, wid


def _hist_accum(hist_ref, idx_ref, val16):
    for l in range(CHUNK // 16):
        idx16 = idx_ref[pl.ds(l * 16, 16)]
        plsc.addupdate_scatter(hist_ref, [idx16], val16)


def _k1_body(x_hbm, idxi_hbm, idxj_hbm, zeros2d_hbm, zeros1d_hbm,
             gj_hbm, ssum_hbm, hist_hbm,
             idxi0, idxi1, idxj0, idxj1, rj0, rj1,
             hist_v, ssum_sh, sj0, sj1):
    cid, sid, wid = _worker_ids()
    idxi = (idxi0, idxi1)
    idxj = (idxj0, idxj1)
    rj = (rj0, rj1)
    sj = (sj0, sj1)

    @pl.when(sid == 0)
    def _():
        pltpu.sync_copy(zeros2d_hbm, ssum_sh)

    pltpu.sync_copy(zeros1d_hbm, hist_v)
    plsc.subcore_barrier()

    ones16 = jnp.ones((16,), jnp.float32)

    def load_and_fire(t, b):
        c = wid + NW * t

        @pl.when(c < NCHUNKS)
        def _():
            base = c * CHUNK
            pltpu.sync_copy(idxi_hbm.at[pl.ds(base, CHUNK)], idxi[b])
            pltpu.sync_copy(idxj_hbm.at[pl.ds(base, CHUNK)], idxj[b])
            pltpu.async_copy(x_hbm.at[idxj[b]], rj[b], sj[b])

    for b in range(2):
        load_and_fire(jnp.int32(b), b)

    @pl.loop(0, NT, step=2)
    def _(t0):
        for b in range(2):
            t = t0 + b
            c = wid + NW * t

            @pl.when(c < NCHUNKS)
            def _():
                base = c * CHUNK
                pltpu.make_async_copy(x_hbm.at[idxj[b]], rj[b], sj[b]).wait()
                pltpu.sync_copy(rj[b], gj_hbm.at[pl.ds(base, CHUNK)])
                pltpu.sync_copy(rj[b], ssum_sh.at[idxi[b]], add=True)
                _hist_accum(hist_v, idxi[b], ones16)

            load_and_fire(t + 2, b)

    pltpu.sync_copy(hist_v, hist_hbm.at[wid])
    plsc.subcore_barrier()

    @pl.when(sid == 0)
    def _():
        pltpu.sync_copy(ssum_sh, ssum_hbm.at[cid])


def _k2_body(x_hbm, b_hbm, idxi_hbm, gi_hbm, bi_hbm,
             idxi0, idxi1, rx0, rx1, rb0, rb1, sx0, sx1, sb0, sb1):
    _, _, wid = _worker_ids()
    idxi = (idxi0, idxi1)
    rx = (rx0, rx1)
    rb = (rb0, rb1)
    sx = (sx0, sx1)
    sb = (sb0, sb1)

    def load_and_fire(t, b):
        c = wid + NW * t

        @pl.when(c < NCHUNKS)
        def _():
            base = c * CHUNK
            pltpu.sync_copy(idxi_hbm.at[pl.ds(base, CHUNK)], idxi[b])
            pltpu.async_copy(x_hbm.at[idxi[b]], rx[b], sx[b])
            pltpu.async_copy(b_hbm.at[idxi[b]], rb[b], sb[b])

    for b in range(2):
        load_and_fire(jnp.int32(b), b)

    @pl.loop(0, NT, step=2)
    def _(t0):
        for b in range(2):
            t = t0 + b
            c = wid + NW * t

            @pl.when(c < NCHUNKS)
            def _():
                base = c * CHUNK
                pltpu.make_async_copy(x_hbm.at[idxi[b]], rx[b], sx[b]).wait()
                pltpu.make_async_copy(b_hbm.at[idxi[b]], rb[b], sb[b]).wait()
                pltpu.sync_copy(rx[b], gi_hbm.at[pl.ds(base, CHUNK)])
                pltpu.sync_copy(rb[b], bi_hbm.at[pl.ds(base, CHUNK)])

            load_and_fire(t + 2, b)


def _k3_body(w_hbm, ex_hbm, idxi_hbm, zeros2d_hbm, zeros1d_hbm,
             wsum_hbm, dhist_hbm,
             idxi0, idxi1, r0, r1, ex0, ex1, dhist_v, wsum_sh, s0, s1):
    cid, sid, wid = _worker_ids()
    idxi = (idxi0, idxi1)
    rows = (r0, r1)
    exv = (ex0, ex1)
    sem = (s0, s1)

    @pl.when(sid == 0)
    def _():
        pltpu.sync_copy(zeros2d_hbm, wsum_sh)

    pltpu.sync_copy(zeros1d_hbm, dhist_v)
    plsc.subcore_barrier()

    def load_and_fire(t, b):
        c = wid + NW * t

        @pl.when(c < NCHUNKS)
        def _():
            base = c * CHUNK
            pltpu.sync_copy(idxi_hbm.at[pl.ds(base, CHUNK)], idxi[b])
            pltpu.sync_copy(ex_hbm.at[pl.ds(base, CHUNK)], exv[b])
            pltpu.async_copy(w_hbm.at[pl.ds(base, CHUNK)], rows[b], sem[b])

    for b in range(2):
        load_and_fire(jnp.int32(b), b)

    @pl.loop(0, NT, step=2)
    def _(t0):
        for b in range(2):
            t = t0 + b
            c = wid + NW * t

            @pl.when(c < NCHUNKS)
            def _():
                pltpu.make_async_copy(
                    w_hbm.at[pl.ds(c * CHUNK, CHUNK)], rows[b], sem[b]).wait()
                pltpu.sync_copy(rows[b], wsum_sh.at[idxi[b]], add=True)
                for l in range(CHUNK // 16):
                    idx16 = idxi[b][pl.ds(l * 16, 16)]
                    ex16 = exv[b][pl.ds(l * 16, 16)]
                    plsc.addupdate_scatter(dhist_v, [idx16], ex16)

            load_and_fire(t + 2, b)

    pltpu.sync_copy(dhist_v, dhist_hbm.at[wid])
    plsc.subcore_barrier()

    @pl.when(sid == 0)
    def _():
        pltpu.sync_copy(wsum_sh, wsum_hbm.at[cid])


_f32 = jnp.float32


def _sc_gather_scatter_pass1(x, idx_i, idx_j):
    zeros2d = jnp.zeros((N, F), _f32)
    zeros1d = jnp.zeros((N,), _f32)
    k1 = pl.kernel(
        _k1_body,
        out_type=(
            jax.ShapeDtypeStruct((E, F), _f32),
            jax.ShapeDtypeStruct((NC, N, F), _f32),
            jax.ShapeDtypeStruct((NW, N), _f32),
        ),
        scratch_types=(
            [pltpu.VMEM((CHUNK,), jnp.int32)] * 4
            + [pltpu.VMEM((CHUNK, F), _f32)] * 2
            + [pltpu.VMEM((N,), _f32), pltpu.VMEM_SHARED((N, F), _f32)]
            + [pltpu.SemaphoreType.DMA] * 2
        ),
        **_SC_MESH,
    )
    return k1(x, idx_i, idx_j, zeros2d, zeros1d)


def _sc_gather_xi_b(x, b_tab, idx_i):
    k2 = pl.kernel(
        _k2_body,
        out_type=(
            jax.ShapeDtypeStruct((E, F), _f32),
            jax.ShapeDtypeStruct((E, F), _f32),
        ),
        scratch_types=(
            [pltpu.VMEM((CHUNK,), jnp.int32)] * 2
            + [pltpu.VMEM((CHUNK, F), _f32)] * 4
            + [pltpu.SemaphoreType.DMA] * 4
        ),
        **_SC_MESH,
    )
    return k2(x, b_tab, idx_i)


def _sc_scatter_pass3(weighted, e_x, idx_i):
    zeros2d = jnp.zeros((N, F), _f32)
    zeros1d = jnp.zeros((N,), _f32)
    k3 = pl.kernel(
        _k3_body,
        out_type=(
            jax.ShapeDtypeStruct((NC, N, F), _f32),
            jax.ShapeDtypeStruct((NW, N), _f32),
        ),
        scratch_types=(
            [pltpu.VMEM((CHUNK,), jnp.int32)] * 2
            + [pltpu.VMEM((CHUNK, F), _f32)] * 2
            + [pltpu.VMEM((CHUNK,), _f32)] * 2
            + [pltpu.VMEM((N,), _f32), pltpu.VMEM_SHARED((N, F), _f32)]
            + [pltpu.SemaphoreType.DMA] * 2
        ),
        **_SC_MESH,
    )
    return k3(weighted, e_x, idx_i, zeros2d, zeros1d)


_TROWS = 1000


def _t1_body(x_ref, ssum_ref, histt_ref, w1b_ref, b_ref, counts_ref):
    counts = jnp.sum(histt_ref[...], axis=1, keepdims=True)
    cmax = jnp.maximum(counts, 1.0)
    ga = (counts * x_ref[...] + ssum_ref[0] + ssum_ref[1]) * 0.5 / cmax
    b_ref[...] = jnp.dot(ga, w1b_ref[...], preferred_element_type=_f32)
    counts_ref[...] = counts


def _tc_combine(x, ssum_p, hist, w1b):
    histt = hist.T
    grid = (N // _TROWS,)
    return pl.pallas_call(
        _t1_body,
        grid=grid,
        in_specs=[
            pl.BlockSpec((_TROWS, F), lambda b: (b, 0)),
            pl.BlockSpec((NC, _TROWS, F), lambda b: (0, b, 0)),
            pl.BlockSpec((_TROWS, NW), lambda b: (b, 0)),
            pl.BlockSpec((F, F), lambda b: (0, 0)),
        ],
        out_specs=[
            pl.BlockSpec((_TROWS, F), lambda b: (b, 0)),
            pl.BlockSpec((_TROWS, 1), lambda b: (b, 0)),
        ],
        out_shape=[
            jax.ShapeDtypeStruct((N, F), _f32),
            jax.ShapeDtypeStruct((N, 1), _f32),
        ],
    )(x, ssum_p, histt, w1b)


_EBLK = 1280
_BF = jnp.bfloat16


def _t2_body(gi_ref, gj_ref, bi_ref,
             w1m_ref, eb1_ref, gb1_ref, w2m_ref, b2m_ref, w3m_ref,
             eb3_ref, gb3_ref,
             sw1a_ref, sw1b_ref, sw1c_ref, sb1_ref, sw2_ref, sb2_ref,
             sw3_ref, sb3_ref,
             w_ref, ex_ref):
    gi = gi_ref[...]
    gj = gj_ref[...]
    gi16 = gi.astype(_BF)
    gj16 = gj.astype(_BF)
    nm16 = ((gi + gj) * 0.5).astype(_BF)

    def mm(a, b):
        return jnp.dot(a, b, preferred_element_type=_f32)

    t = mm(nm16, w1m_ref[...])
    h = jnp.maximum(t[:, :128] + eb1_ref[...], 0.0)
    g = jnp.maximum(t[:, 128:] + bi_ref[...] + gb1_ref[...], 0.0)
    hg = jnp.concatenate([h, g], axis=1).astype(_BF)
    u = jnp.maximum(mm(hg, w2m_ref[...]) + b2m_ref[...], 0.0)
    v = mm(u.astype(_BF), w3m_ref[...])
    eij = v[:, 0:1] + eb3_ref[...]
    edge = v[:, 1:2] + gb3_ref[...]
    e_x = jnp.exp(eij)

    s = jnp.maximum(mm(gi16, sw1a_ref[...]) + mm(gj16, sw1b_ref[...])
                    + edge * sw1c_ref[...] + sb1_ref[...], 0.0)
    s = jnp.maximum(mm(s.astype(_BF), sw2_ref[...]) + sb2_ref[...], 0.0)
    m = mm(s.astype(_BF), sw3_ref[...]) + sb3_ref[...]

    w_ref[...] = e_x * m
    ex_ref[...] = e_x


def _tc_mlp(gi, gj, bi, weights):
    grid = (E // _EBLK,)
    eblk = lambda b: (b, 0)
    full = lambda b: (0, 0)
    wspecs = [pl.BlockSpec(w.shape, full) for w in weights]
    return pl.pallas_call(
        _t2_body,
        grid=grid,
        in_specs=[pl.BlockSpec((_EBLK, F), eblk)] * 3 + wspecs,
        out_specs=[
            pl.BlockSpec((_EBLK, F), eblk),
            pl.BlockSpec((_EBLK, 1), eblk),
        ],
        out_shape=[
            jax.ShapeDtypeStruct((E, F), _f32),
            jax.ShapeDtypeStruct((E, 1), _f32),
        ],
    )(gi, gj, bi, *weights)


def _t3_body(x_ref, wsum_ref, dhistt_ref, counts_ref, out_ref):
    dsum = jnp.sum(dhistt_ref[...], axis=1, keepdims=True)
    cmax = jnp.maximum(counts_ref[...], 1.0)
    agg = (wsum_ref[0] + wsum_ref[1]) / ((dsum + 1e-9) * cmax)
    out_ref[...] = jnp.maximum(x_ref[...] + agg, 0.0)


def _tc_final(x, wsum_p, dhist, counts):
    dhistt = dhist.T
    grid = (N // _TROWS,)
    return pl.pallas_call(
        _t3_body,
        grid=grid,
        in_specs=[
            pl.BlockSpec((_TROWS, F), lambda b: (b, 0)),
            pl.BlockSpec((NC, _TROWS, F), lambda b: (0, b, 0)),
            pl.BlockSpec((_TROWS, NW), lambda b: (b, 0)),
            pl.BlockSpec((_TROWS, 1), lambda b: (b, 0)),
        ],
        out_specs=pl.BlockSpec((_TROWS, F), lambda b: (b, 0)),
        out_shape=jax.ShapeDtypeStruct((N, F), _f32),
    )(x, wsum_p, dhistt, counts)


def kernel(x, edge_index, e,
           enet_W1, enet_b1, enet_W2, enet_b2, enet_W3, enet_b3,
           edgenet_W1, edgenet_b1, edgenet_W2, edgenet_b2, edgenet_W3, edgenet_b3,
           snet_W1, snet_b1, snet_W2, snet_b2, snet_W3, snet_b3):
    idx_i = edge_index[0]
    idx_j = edge_index[1]

    gj, ssum_p, hist = _sc_gather_scatter_pass1(x, idx_i, idx_j)

    b_tab, counts = _tc_combine(x, ssum_p, hist, edgenet_W1[F:])

    gi, bi = _sc_gather_xi_b(x, b_tab, idx_i)

    w1m = jnp.concatenate([enet_W1, edgenet_W1[:F]], axis=1).astype(_BF)
    z128_64 = jnp.zeros((F, 64), _f32)
    w2m = jnp.concatenate([
        jnp.concatenate([enet_W2, z128_64], axis=1),
        jnp.concatenate([z128_64, edgenet_W2], axis=1),
    ], axis=0).astype(_BF)
    b2m = jnp.concatenate([enet_b2, edgenet_b2]).reshape(1, -1)
    z64_1 = jnp.zeros((64, 1), _f32)
    w3m = jnp.concatenate([
        jnp.concatenate([enet_W3, z64_1], axis=1),
        jnp.concatenate([z64_1, edgenet_W3], axis=1),
    ], axis=0).astype(_BF)
    weights = (
        w1m, enet_b1.reshape(1, -1), edgenet_b1.reshape(1, -1),
        w2m, b2m, w3m,
        enet_b3.reshape(1, -1), edgenet_b3.reshape(1, -1),
        snet_W1[:F].astype(_BF), snet_W1[F:2 * F].astype(_BF),
        snet_W1[2 * F:], snet_b1.reshape(1, -1),
        snet_W2.astype(_BF), snet_b2.reshape(1, -1),
        snet_W3.astype(_BF), snet_b3.reshape(1, -1),
    )
    weighted, e_x = _tc_mlp(gi, gj, bi, weights)

    wsum_p, dhist = _sc_scatter_pass3(weighted, e_x[:, 0], idx_i)

    return _tc_final(x, wsum_p, dhist, counts)


# speedup vs baseline: 1.1807x; 1.0803x over previous
import functools

import jax
import jax.numpy as jnp
import numpy as np
from jax import lax
from jax.experimental import pallas as pl
from jax.experimental.pallas import tpu as pltpu
from jax.experimental.pallas import tpu_sc as plsc

N = 10000
E = 320000
F = 128

NC = 2
NS = 16
NW = NC * NS
CHUNK = 128
NCHUNKS = E // CHUNK
BASE_CH = NCHUNKS // NW
REM_CH = NCHUNKS - BASE_CH * NW

_SC_MESH = dict(
    mesh=plsc.VectorSubcoreMesh(core_axis_name="c", subcore_axis_name="s"),
    compiler_params=pltpu.CompilerParams(needs_layout_passes=False),
)


NT = 80


def _worker_ids():
    cid = lax.axis_index("c")
    sid = lax.axis_index("s")
    wid = sid * NC + cid
    return cid, sid, wid


def _hist_accum(hist_ref, idx_ref, val16):
    for l in range(CHUNK // 16):
        idx16 = idx_ref[pl.ds(l * 16, 16)]
        plsc.addupdate_scatter(hist_ref, [idx16], val16)


def _k1_body(x_hbm, idxi_hbm, idxj_hbm, zeros2d_hbm, zeros1d_hbm,
             gj_hbm, ssum_hbm, hist_hbm,
             idxi0, idxi1, idxj0, idxj1, rj0, rj1,
             hist_v, ssum_sh, sj0, sj1):
    cid, sid, wid = _worker_ids()
    idxi = (idxi0, idxi1)
    idxj = (idxj0, idxj1)
    rj = (rj0, rj1)
    sj = (sj0, sj1)

    @pl.when(sid == 0)
    def _():
        pltpu.sync_copy(zeros2d_hbm, ssum_sh)

    pltpu.sync_copy(zeros1d_hbm, hist_v)
    plsc.subcore_barrier()

    ones16 = jnp.ones((16,), jnp.float32)

    def load_and_fire(t, b):
        c = wid + NW * t

        @pl.when(c < NCHUNKS)
        def _():
            base = c * CHUNK
            pltpu.sync_copy(idxi_hbm.at[pl.ds(base, CHUNK)], idxi[b])
            pltpu.sync_copy(idxj_hbm.at[pl.ds(base, CHUNK)], idxj[b])
            pltpu.async_copy(x_hbm.at[idxj[b]], rj[b], sj[b])

    for b in range(2):
        load_and_fire(jnp.int32(b), b)

    @pl.loop(0, NT, step=2)
    def _(t0):
        for b in range(2):
            t = t0 + b
            c = wid + NW * t

            @pl.when(c < NCHUNKS)
            def _():
                base = c * CHUNK
                pltpu.make_async_copy(x_hbm.at[idxj[b]], rj[b], sj[b]).wait()
                pltpu.sync_copy(rj[b], gj_hbm.at[pl.ds(base, CHUNK)])
                pltpu.sync_copy(rj[b], ssum_sh.at[idxi[b]], add=True)
                _hist_accum(hist_v, idxi[b], ones16)

            load_and_fire(t + 2, b)

    pltpu.sync_copy(hist_v, hist_hbm.at[wid])
    plsc.subcore_barrier()

    @pl.when(sid == 0)
    def _():
        pltpu.sync_copy(ssum_sh, ssum_hbm.at[cid])


def _k2_body(x_hbm, b_hbm, idxi_hbm, gi_hbm, bi_hbm,
             idxi0, idxi1, rx0, rx1, rb0, rb1, sx0, sx1, sb0, sb1):
    _, _, wid = _worker_ids()
    idxi = (idxi0, idxi1)
    rx = (rx0, rx1)
    rb = (rb0, rb1)
    sx = (sx0, sx1)
    sb = (sb0, sb1)

    def load_and_fire(t, b):
        c = wid + NW * t

        @pl.when(c < NCHUNKS)
        def _():
            base = c * CHUNK
            pltpu.sync_copy(idxi_hbm.at[pl.ds(base, CHUNK)], idxi[b])
            pltpu.async_copy(x_hbm.at[idxi[b]], rx[b], sx[b])
            pltpu.async_copy(b_hbm.at[idxi[b]], rb[b], sb[b])

    for b in range(2):
        load_and_fire(jnp.int32(b), b)

    @pl.loop(0, NT, step=2)
    def _(t0):
        for b in range(2):
            t = t0 + b
            c = wid + NW * t

            @pl.when(c < NCHUNKS)
            def _():
                base = c * CHUNK
                pltpu.make_async_copy(x_hbm.at[idxi[b]], rx[b], sx[b]).wait()
                pltpu.make_async_copy(b_hbm.at[idxi[b]], rb[b], sb[b]).wait()
                pltpu.sync_copy(rx[b], gi_hbm.at[pl.ds(base, CHUNK)])
                pltpu.sync_copy(rb[b], bi_hbm.at[pl.ds(base, CHUNK)])

            load_and_fire(t + 2, b)


def _k3_body(w_hbm, ex_hbm, idxi_hbm, zeros2d_hbm, zeros1d_hbm,
             wsum_hbm, dhist_hbm,
             idxi0, idxi1, r0, r1, ex0, ex1, dhist_v, wsum_sh, s0, s1):
    cid, sid, wid = _worker_ids()
    idxi = (idxi0, idxi1)
    rows = (r0, r1)
    exv = (ex0, ex1)
    sem = (s0, s1)

    @pl.when(sid == 0)
    def _():
        pltpu.sync_copy(zeros2d_hbm, wsum_sh)

    pltpu.sync_copy(zeros1d_hbm, dhist_v)
    plsc.subcore_barrier()

    def load_and_fire(t, b):
        c = wid + NW * t

        @pl.when(c < NCHUNKS)
        def _():
            base = c * CHUNK
            pltpu.sync_copy(idxi_hbm.at[pl.ds(base, CHUNK)], idxi[b])
            pltpu.sync_copy(ex_hbm.at[pl.ds(base, CHUNK)], exv[b])
            pltpu.async_copy(w_hbm.at[pl.ds(base, CHUNK)], rows[b], sem[b])

    for b in range(2):
        load_and_fire(jnp.int32(b), b)

    @pl.loop(0, NT, step=2)
    def _(t0):
        for b in range(2):
            t = t0 + b
            c = wid + NW * t

            @pl.when(c < NCHUNKS)
            def _():
                pltpu.make_async_copy(
                    w_hbm.at[pl.ds(c * CHUNK, CHUNK)], rows[b], sem[b]).wait()
                pltpu.sync_copy(rows[b], wsum_sh.at[idxi[b]], add=True)
                for l in range(CHUNK // 16):
                    idx16 = idxi[b][pl.ds(l * 16, 16)]
                    ex16 = exv[b][pl.ds(l * 16, 16)]
                    plsc.addupdate_scatter(dhist_v, [idx16], ex16)

            load_and_fire(t + 2, b)

    pltpu.sync_copy(dhist_v, dhist_hbm.at[wid])
    plsc.subcore_barrier()

    @pl.when(sid == 0)
    def _():
        pltpu.sync_copy(wsum_sh, wsum_hbm.at[cid])


_f32 = jnp.float32


def _sc_gather_scatter_pass1(x, idx_i, idx_j):
    zeros2d = jnp.zeros((N, F), _f32)
    zeros1d = jnp.zeros((N,), _f32)
    k1 = pl.kernel(
        _k1_body,
        out_type=(
            jax.ShapeDtypeStruct((E, F), _f32),
            jax.ShapeDtypeStruct((NC, N, F), _f32),
            jax.ShapeDtypeStruct((NW, N), _f32),
        ),
        scratch_types=(
            [pltpu.VMEM((CHUNK,), jnp.int32)] * 4
            + [pltpu.VMEM((CHUNK, F), _f32)] * 2
            + [pltpu.VMEM((N,), _f32), pltpu.VMEM_SHARED((N, F), _f32)]
            + [pltpu.SemaphoreType.DMA] * 2
        ),
        **_SC_MESH,
    )
    return k1(x, idx_i, idx_j, zeros2d, zeros1d)


def _sc_gather_xi_b(x, b_tab, idx_i):
    k2 = pl.kernel(
        _k2_body,
        out_type=(
            jax.ShapeDtypeStruct((E, F), _f32),
            jax.ShapeDtypeStruct((E, F), _f32),
        ),
        scratch_types=(
            [pltpu.VMEM((CHUNK,), jnp.int32)] * 2
            + [pltpu.VMEM((CHUNK, F), _f32)] * 4
            + [pltpu.SemaphoreType.DMA] * 4
        ),
        **_SC_MESH,
    )
    return k2(x, b_tab, idx_i)


def _sc_scatter_pass3(weighted, e_x, idx_i):
    zeros2d = jnp.zeros((N, F), _f32)
    zeros1d = jnp.zeros((N,), _f32)
    k3 = pl.kernel(
        _k3_body,
        out_type=(
            jax.ShapeDtypeStruct((NC, N, F), _f32),
            jax.ShapeDtypeStruct((NW, N), _f32),
        ),
        scratch_types=(
            [pltpu.VMEM((CHUNK,), jnp.int32)] * 2
            + [pltpu.VMEM((CHUNK, F), _f32)] * 2
            + [pltpu.VMEM((CHUNK,), _f32)] * 2
            + [pltpu.VMEM((N,), _f32), pltpu.VMEM_SHARED((N, F), _f32)]
            + [pltpu.SemaphoreType.DMA] * 2
        ),
        **_SC_MESH,
    )
    return k3(weighted, e_x, idx_i, zeros2d, zeros1d)


_TROWS = 1000


def _t1_body(x_ref, ssum_ref, histt_ref, w1b_ref, b_ref, counts_ref):
    counts = jnp.sum(histt_ref[...], axis=1, keepdims=True)
    cmax = jnp.maximum(counts, 1.0)
    ga = (counts * x_ref[...] + ssum_ref[0] + ssum_ref[1]) * 0.5 / cmax
    b_ref[...] = jnp.dot(ga, w1b_ref[...], preferred_element_type=_f32)
    counts_ref[...] = counts


def _tc_combine(x, ssum_p, hist, w1b):
    histt = hist.T
    grid = (N // _TROWS,)
    return pl.pallas_call(
        _t1_body,
        grid=grid,
        in_specs=[
            pl.BlockSpec((_TROWS, F), lambda b: (b, 0)),
            pl.BlockSpec((NC, _TROWS, F), lambda b: (0, b, 0)),
            pl.BlockSpec((_TROWS, NW), lambda b: (b, 0)),
            pl.BlockSpec((F, F), lambda b: (0, 0)),
        ],
        out_specs=[
            pl.BlockSpec((_TROWS, F), lambda b: (b, 0)),
            pl.BlockSpec((_TROWS, 1), lambda b: (b, 0)),
        ],
        out_shape=[
            jax.ShapeDtypeStruct((N, F), _f32),
            jax.ShapeDtypeStruct((N, 1), _f32),
        ],
    )(x, ssum_p, histt, w1b)


_EBLK = 2560
_BF = jnp.bfloat16


def _t2_body(gi_ref, gj_ref, bi_ref,
             w1m_ref, eb1_ref, gb1_ref, w2m_ref, b2m_ref, w3m_ref,
             eb3_ref, gb3_ref,
             sw1a_ref, sw1b_ref, sw1c_ref, sb1_ref, sw2_ref, sb2_ref,
             sw3_ref, sb3_ref,
             w_ref, ex_ref):
    gi = gi_ref[...]
    gj = gj_ref[...]
    gi16 = gi.astype(_BF)
    gj16 = gj.astype(_BF)
    nm16 = ((gi + gj) * 0.5).astype(_BF)

    def mm(a, b):
        return jnp.dot(a, b, preferred_element_type=_f32)

    t = mm(nm16, w1m_ref[...])
    h = jnp.maximum(t[:, :128] + eb1_ref[...], 0.0)
    g = jnp.maximum(t[:, 128:] + bi_ref[...] + gb1_ref[...], 0.0)
    hg = jnp.concatenate([h, g], axis=1).astype(_BF)
    u = jnp.maximum(mm(hg, w2m_ref[...]) + b2m_ref[...], 0.0)
    v = mm(u.astype(_BF), w3m_ref[...])
    eij = v[:, 0:1] + eb3_ref[...]
    edge = v[:, 1:2] + gb3_ref[...]
    e_x = jnp.exp(eij)

    s = jnp.maximum(mm(gi16, sw1a_ref[...]) + mm(gj16, sw1b_ref[...])
                    + edge * sw1c_ref[...] + sb1_ref[...], 0.0)
    s = jnp.maximum(mm(s.astype(_BF), sw2_ref[...]) + sb2_ref[...], 0.0)
    m = mm(s.astype(_BF), sw3_ref[...]) + sb3_ref[...]

    w_ref[...] = e_x * m
    ex_ref[...] = e_x


def _tc_mlp(gi, gj, bi, weights):
    grid = (E // _EBLK,)
    eblk = lambda b: (b, 0)
    full = lambda b: (0, 0)
    wspecs = [pl.BlockSpec(w.shape, full) for w in weights]
    return pl.pallas_call(
        _t2_body,
        grid=grid,
        in_specs=[pl.BlockSpec((_EBLK, F), eblk)] * 3 + wspecs,
        out_specs=[
            pl.BlockSpec((_EBLK, F), eblk),
            pl.BlockSpec((_EBLK, 1), eblk),
        ],
        out_shape=[
            jax.ShapeDtypeStruct((E, F), _f32),
            jax.ShapeDtypeStruct((E, 1), _f32),
        ],
    )(gi, gj, bi, *weights)


def _t3_body(x_ref, wsum_ref, dhistt_ref, counts_ref, out_ref):
    dsum = jnp.sum(dhistt_ref[...], axis=1, keepdims=True)
    cmax = jnp.maximum(counts_ref[...], 1.0)
    agg = (wsum_ref[0] + wsum_ref[1]) / ((dsum + 1e-9) * cmax)
    out_ref[...] = jnp.maximum(x_ref[...] + agg, 0.0)


def _tc_final(x, wsum_p, dhist, counts):
    dhistt = dhist.T
    grid = (N // _TROWS,)
    return pl.pallas_call(
        _t3_body,
        grid=grid,
        in_specs=[
            pl.BlockSpec((_TROWS, F), lambda b: (b, 0)),
            pl.BlockSpec((NC, _TROWS, F), lambda b: (0, b, 0)),
            pl.BlockSpec((_TROWS, NW), lambda b: (b, 0)),
            pl.BlockSpec((_TROWS, 1), lambda b: (b, 0)),
        ],
        out_specs=pl.BlockSpec((_TROWS, F), lambda b: (b, 0)),
        out_shape=jax.ShapeDtypeStruct((N, F), _f32),
    )(x, wsum_p, dhistt, counts)


def kernel(x, edge_index, e,
           enet_W1, enet_b1, enet_W2, enet_b2, enet_W3, enet_b3,
           edgenet_W1, edgenet_b1, edgenet_W2, edgenet_b2, edgenet_W3, edgenet_b3,
           snet_W1, snet_b1, snet_W2, snet_b2, snet_W3, snet_b3):
    idx_i = edge_index[0]
    idx_j = edge_index[1]

    gj, ssum_p, hist = _sc_gather_scatter_pass1(x, idx_i, idx_j)

    b_tab, counts = _tc_combine(x, ssum_p, hist, edgenet_W1[F:])

    gi, bi = _sc_gather_xi_b(x, b_tab, idx_i)

    w1m = jnp.concatenate([enet_W1, edgenet_W1[:F]], axis=1).astype(_BF)
    z128_64 = jnp.zeros((F, 64), _f32)
    w2m = jnp.concatenate([
        jnp.concatenate([enet_W2, z128_64], axis=1),
        jnp.concatenate([z128_64, edgenet_W2], axis=1),
    ], axis=0).astype(_BF)
    b2m = jnp.concatenate([enet_b2, edgenet_b2]).reshape(1, -1)
    z64_1 = jnp.zeros((64, 1), _f32)
    w3m = jnp.concatenate([
        jnp.concatenate([enet_W3, z64_1], axis=1),
        jnp.concatenate([z64_1, edgenet_W3], axis=1),
    ], axis=0).astype(_BF)
    weights = (
        w1m, enet_b1.reshape(1, -1), edgenet_b1.reshape(1, -1),
        w2m, b2m, w3m,
        enet_b3.reshape(1, -1), edgenet_b3.reshape(1, -1),
        snet_W1[:F].astype(_BF), snet_W1[F:2 * F].astype(_BF),
        snet_W1[2 * F:], snet_b1.reshape(1, -1),
        snet_W2.astype(_BF), snet_b2.reshape(1, -1),
        snet_W3.astype(_BF), snet_b3.reshape(1, -1),
    )
    weighted, e_x = _tc_mlp(gi, gj, bi, weights)

    wsum_p, dhist = _sc_scatter_pass3(weighted, e_x[:, 0], idx_i)

    return _tc_final(x, wsum_p, dhist, counts)


# speedup vs baseline: 1.2170x; 1.0308x over previous
import functools

import jax
import jax.numpy as jnp
import numpy as np
from jax import lax
from jax.experimental import pallas as pl
from jax.experimental.pallas import tpu as pltpu
from jax.experimental.pallas import tpu_sc as plsc

N = 10000
E = 320000
F = 128

NC = 2
NS = 16
NW = NC * NS
CHUNK = 128
NCHUNKS = E // CHUNK
BASE_CH = NCHUNKS // NW
REM_CH = NCHUNKS - BASE_CH * NW

_SC_MESH = dict(
    mesh=plsc.VectorSubcoreMesh(core_axis_name="c", subcore_axis_name="s"),
    compiler_params=pltpu.CompilerParams(needs_layout_passes=False),
)


NT = 80


def _worker_ids():
    cid = lax.axis_index("c")
    sid = lax.axis_index("s")
    wid = sid * NC + cid
    return cid, sid, wid


def _hist_accum(hist_ref, idx_ref, val16):
    for l in range(CHUNK // 16):
        idx16 = idx_ref[pl.ds(l * 16, 16)]
        plsc.addupdate_scatter(hist_ref, [idx16], val16)


def _k1_body(x_hbm, idxi_hbm, idxj_hbm, zeros2d_hbm, zeros1d_hbm,
             gj_hbm, ssum_hbm, hist_hbm,
             idxi0, idxi1, idxj0, idxj1, rj0, rj1,
             hist_v, ssum_sh, sj0, sj1):
    cid, sid, wid = _worker_ids()
    idxi = (idxi0, idxi1)
    idxj = (idxj0, idxj1)
    rj = (rj0, rj1)
    sj = (sj0, sj1)

    @pl.when(sid == 0)
    def _():
        pltpu.sync_copy(zeros2d_hbm, ssum_sh)

    pltpu.sync_copy(zeros1d_hbm, hist_v)
    plsc.subcore_barrier()

    ones16 = jnp.ones((16,), jnp.float32)

    def load_and_fire(t, b):
        c = wid + NW * t

        @pl.when(c < NCHUNKS)
        def _():
            base = c * CHUNK
            pltpu.sync_copy(idxi_hbm.at[pl.ds(base, CHUNK)], idxi[b])
            pltpu.sync_copy(idxj_hbm.at[pl.ds(base, CHUNK)], idxj[b])
            pltpu.async_copy(x_hbm.at[idxj[b]], rj[b], sj[b])

    for b in range(2):
        load_and_fire(jnp.int32(b), b)

    @pl.loop(0, NT, step=2)
    def _(t0):
        for b in range(2):
            t = t0 + b
            c = wid + NW * t

            @pl.when(c < NCHUNKS)
            def _():
                base = c * CHUNK
                pltpu.make_async_copy(x_hbm.at[idxj[b]], rj[b], sj[b]).wait()
                pltpu.sync_copy(rj[b], gj_hbm.at[pl.ds(base, CHUNK)])
                pltpu.sync_copy(rj[b], ssum_sh.at[idxi[b]], add=True)
                _hist_accum(hist_v, idxi[b], ones16)

            load_and_fire(t + 2, b)

    pltpu.sync_copy(hist_v, hist_hbm.at[wid])
    plsc.subcore_barrier()

    @pl.when(sid == 0)
    def _():
        pltpu.sync_copy(ssum_sh, ssum_hbm.at[cid])


def _k2_body(x_hbm, b_hbm, idxi_hbm, gi_hbm, bi_hbm,
             idxi0, idxi1, rx0, rx1, rb0, rb1, sx0, sx1, sb0, sb1):
    _, _, wid = _worker_ids()
    idxi = (idxi0, idxi1)
    rx = (rx0, rx1)
    rb = (rb0, rb1)
    sx = (sx0, sx1)
    sb = (sb0, sb1)

    def load_and_fire(t, b):
        c = wid + NW * t

        @pl.when(c < NCHUNKS)
        def _():
            base = c * CHUNK
            pltpu.sync_copy(idxi_hbm.at[pl.ds(base, CHUNK)], idxi[b])
            pltpu.async_copy(x_hbm.at[idxi[b]], rx[b], sx[b])
            pltpu.async_copy(b_hbm.at[idxi[b]], rb[b], sb[b])

    for b in range(2):
        load_and_fire(jnp.int32(b), b)

    @pl.loop(0, NT, step=2)
    def _(t0):
        for b in range(2):
            t = t0 + b
            c = wid + NW * t

            @pl.when(c < NCHUNKS)
            def _():
                base = c * CHUNK
                pltpu.make_async_copy(x_hbm.at[idxi[b]], rx[b], sx[b]).wait()
                pltpu.make_async_copy(b_hbm.at[idxi[b]], rb[b], sb[b]).wait()
                pltpu.sync_copy(rx[b], gi_hbm.at[pl.ds(base, CHUNK)])
                pltpu.sync_copy(rb[b], bi_hbm.at[pl.ds(base, CHUNK)])

            load_and_fire(t + 2, b)


def _k3_body(w_hbm, ex_hbm, idxi_hbm, zeros2d_hbm, zeros1d_hbm,
             wsum_hbm, dhist_hbm,
             idxi0, idxi1, r0, r1, ex0, ex1, dhist_v, wsum_sh, s0, s1):
    cid, sid, wid = _worker_ids()
    idxi = (idxi0, idxi1)
    rows = (r0, r1)
    exv = (ex0, ex1)
    sem = (s0, s1)

    @pl.when(sid == 0)
    def _():
        pltpu.sync_copy(zeros2d_hbm, wsum_sh)

    pltpu.sync_copy(zeros1d_hbm, dhist_v)
    plsc.subcore_barrier()

    def load_and_fire(t, b):
        c = wid + NW * t

        @pl.when(c < NCHUNKS)
        def _():
            base = c * CHUNK
            pltpu.sync_copy(idxi_hbm.at[pl.ds(base, CHUNK)], idxi[b])
            pltpu.sync_copy(ex_hbm.at[pl.ds(base, CHUNK)], exv[b])
            pltpu.async_copy(w_hbm.at[pl.ds(base, CHUNK)], rows[b], sem[b])

    for b in range(2):
        load_and_fire(jnp.int32(b), b)

    @pl.loop(0, NT, step=2)
    def _(t0):
        for b in range(2):
            t = t0 + b
            c = wid + NW * t

            @pl.when(c < NCHUNKS)
            def _():
                pltpu.make_async_copy(
                    w_hbm.at[pl.ds(c * CHUNK, CHUNK)], rows[b], sem[b]).wait()
                pltpu.sync_copy(rows[b], wsum_sh.at[idxi[b]], add=True)
                for l in range(CHUNK // 16):
                    idx16 = idxi[b][pl.ds(l * 16, 16)]
                    ex16 = exv[b][pl.ds(l * 16, 16)]
                    plsc.addupdate_scatter(dhist_v, [idx16], ex16)

            load_and_fire(t + 2, b)

    pltpu.sync_copy(dhist_v, dhist_hbm.at[wid])
    plsc.subcore_barrier()

    @pl.when(sid == 0)
    def _():
        pltpu.sync_copy(wsum_sh, wsum_hbm.at[cid])


_f32 = jnp.float32


def _sc_gather_scatter_pass1(x, idx_i, idx_j):
    zeros2d = jnp.zeros((N, F), _f32)
    zeros1d = jnp.zeros((N,), _f32)
    k1 = pl.kernel(
        _k1_body,
        out_type=(
            jax.ShapeDtypeStruct((E, F), _f32),
            jax.ShapeDtypeStruct((NC, N, F), _f32),
            jax.ShapeDtypeStruct((NW, N), _f32),
        ),
        scratch_types=(
            [pltpu.VMEM((CHUNK,), jnp.int32)] * 4
            + [pltpu.VMEM((CHUNK, F), _f32)] * 2
            + [pltpu.VMEM((N,), _f32), pltpu.VMEM_SHARED((N, F), _f32)]
            + [pltpu.SemaphoreType.DMA] * 2
        ),
        **_SC_MESH,
    )
    return k1(x, idx_i, idx_j, zeros2d, zeros1d)


def _sc_gather_xi_b(x, b_tab, idx_i):
    k2 = pl.kernel(
        _k2_body,
        out_type=(
            jax.ShapeDtypeStruct((E, F), _f32),
            jax.ShapeDtypeStruct((E, F), _f32),
        ),
        scratch_types=(
            [pltpu.VMEM((CHUNK,), jnp.int32)] * 2
            + [pltpu.VMEM((CHUNK, F), _f32)] * 4
            + [pltpu.SemaphoreType.DMA] * 4
        ),
        **_SC_MESH,
    )
    return k2(x, b_tab, idx_i)


def _sc_scatter_pass3(weighted, e_x, idx_i):
    zeros2d = jnp.zeros((N, F), _f32)
    zeros1d = jnp.zeros((N,), _f32)
    k3 = pl.kernel(
        _k3_body,
        out_type=(
            jax.ShapeDtypeStruct((NC, N, F), _f32),
            jax.ShapeDtypeStruct((NW, N), _f32),
        ),
        scratch_types=(
            [pltpu.VMEM((CHUNK,), jnp.int32)] * 2
            + [pltpu.VMEM((CHUNK, F), _f32)] * 2
            + [pltpu.VMEM((CHUNK,), _f32)] * 2
            + [pltpu.VMEM((N,), _f32), pltpu.VMEM_SHARED((N, F), _f32)]
            + [pltpu.SemaphoreType.DMA] * 2
        ),
        **_SC_MESH,
    )
    return k3(weighted, e_x, idx_i, zeros2d, zeros1d)


_TROWS = 1000


def _t1_body(x_ref, ssum_ref, histt_ref, w1b_ref, b_ref, counts_ref):
    counts = jnp.sum(histt_ref[...], axis=1, keepdims=True)
    cmax = jnp.maximum(counts, 1.0)
    ga = (counts * x_ref[...] + ssum_ref[0] + ssum_ref[1]) * 0.5 / cmax
    b_ref[...] = jnp.dot(ga, w1b_ref[...], preferred_element_type=_f32)
    counts_ref[...] = counts


def _tc_combine(x, ssum_p, hist, w1b):
    histt = hist.T
    grid = (N // _TROWS,)
    return pl.pallas_call(
        _t1_body,
        grid=grid,
        in_specs=[
            pl.BlockSpec((_TROWS, F), lambda b: (b, 0)),
            pl.BlockSpec((NC, _TROWS, F), lambda b: (0, b, 0)),
            pl.BlockSpec((_TROWS, NW), lambda b: (b, 0)),
            pl.BlockSpec((F, F), lambda b: (0, 0)),
        ],
        out_specs=[
            pl.BlockSpec((_TROWS, F), lambda b: (b, 0)),
            pl.BlockSpec((_TROWS, 1), lambda b: (b, 0)),
        ],
        out_shape=[
            jax.ShapeDtypeStruct((N, F), _f32),
            jax.ShapeDtypeStruct((N, 1), _f32),
        ],
    )(x, ssum_p, histt, w1b)


_EBLK = 4000
_BF = jnp.bfloat16


def _t2_body(gi_ref, gj_ref, bi_ref,
             w1m_ref, eb1_ref, gb1_ref, w2m_ref, b2m_ref, w3m_ref,
             eb3_ref, gb3_ref,
             sw1a_ref, sw1b_ref, sw1c_ref, sb1_ref, sw2_ref, sb2_ref,
             sw3_ref, sb3_ref,
             w_ref, ex_ref):
    gi = gi_ref[...]
    gj = gj_ref[...]
    gi16 = gi.astype(_BF)
    gj16 = gj.astype(_BF)
    nm16 = ((gi + gj) * 0.5).astype(_BF)

    def mm(a, b):
        return jnp.dot(a, b, preferred_element_type=_f32)

    t = mm(nm16, w1m_ref[...])
    h = jnp.maximum(t[:, :128] + eb1_ref[...], 0.0)
    g = jnp.maximum(t[:, 128:] + bi_ref[...] + gb1_ref[...], 0.0)
    hg = jnp.concatenate([h, g], axis=1).astype(_BF)
    u = jnp.maximum(mm(hg, w2m_ref[...]) + b2m_ref[...], 0.0)
    v = mm(u.astype(_BF), w3m_ref[...])
    eij = v[:, 0:1] + eb3_ref[...]
    edge = v[:, 1:2] + gb3_ref[...]
    e_x = jnp.exp(eij)

    s = jnp.maximum(mm(gi16, sw1a_ref[...]) + mm(gj16, sw1b_ref[...])
                    + edge * sw1c_ref[...] + sb1_ref[...], 0.0)
    s = jnp.maximum(mm(s.astype(_BF), sw2_ref[...]) + sb2_ref[...], 0.0)
    m = mm(s.astype(_BF), sw3_ref[...]) + sb3_ref[...]

    w_ref[...] = e_x * m
    ex_ref[...] = e_x


def _tc_mlp(gi, gj, bi, weights):
    grid = (E // _EBLK,)
    eblk = lambda b: (b, 0)
    full = lambda b: (0, 0)
    wspecs = [pl.BlockSpec(w.shape, full) for w in weights]
    return pl.pallas_call(
        _t2_body,
        grid=grid,
        in_specs=[pl.BlockSpec((_EBLK, F), eblk)] * 3 + wspecs,
        out_specs=[
            pl.BlockSpec((_EBLK, F), eblk),
            pl.BlockSpec((_EBLK, 1), eblk),
        ],
        out_shape=[
            jax.ShapeDtypeStruct((E, F), _f32),
            jax.ShapeDtypeStruct((E, 1), _f32),
        ],
    )(gi, gj, bi, *weights)


def _t3_body(x_ref, wsum_ref, dhistt_ref, counts_ref, out_ref):
    dsum = jnp.sum(dhistt_ref[...], axis=1, keepdims=True)
    cmax = jnp.maximum(counts_ref[...], 1.0)
    agg = (wsum_ref[0] + wsum_ref[1]) / ((dsum + 1e-9) * cmax)
    out_ref[...] = jnp.maximum(x_ref[...] + agg, 0.0)


def _tc_final(x, wsum_p, dhist, counts):
    dhistt = dhist.T
    grid = (N // _TROWS,)
    return pl.pallas_call(
        _t3_body,
        grid=grid,
        in_specs=[
            pl.BlockSpec((_TROWS, F), lambda b: (b, 0)),
            pl.BlockSpec((NC, _TROWS, F), lambda b: (0, b, 0)),
            pl.BlockSpec((_TROWS, NW), lambda b: (b, 0)),
            pl.BlockSpec((_TROWS, 1), lambda b: (b, 0)),
        ],
        out_specs=pl.BlockSpec((_TROWS, F), lambda b: (b, 0)),
        out_shape=jax.ShapeDtypeStruct((N, F), _f32),
    )(x, wsum_p, dhistt, counts)


def kernel(x, edge_index, e,
           enet_W1, enet_b1, enet_W2, enet_b2, enet_W3, enet_b3,
           edgenet_W1, edgenet_b1, edgenet_W2, edgenet_b2, edgenet_W3, edgenet_b3,
           snet_W1, snet_b1, snet_W2, snet_b2, snet_W3, snet_b3):
    idx_i = edge_index[0]
    idx_j = edge_index[1]

    gj, ssum_p, hist = _sc_gather_scatter_pass1(x, idx_i, idx_j)

    b_tab, counts = _tc_combine(x, ssum_p, hist, edgenet_W1[F:])

    gi, bi = _sc_gather_xi_b(x, b_tab, idx_i)

    w1m = jnp.concatenate([enet_W1, edgenet_W1[:F]], axis=1).astype(_BF)
    z128_64 = jnp.zeros((F, 64), _f32)
    w2m = jnp.concatenate([
        jnp.concatenate([enet_W2, z128_64], axis=1),
        jnp.concatenate([z128_64, edgenet_W2], axis=1),
    ], axis=0).astype(_BF)
    b2m = jnp.concatenate([enet_b2, edgenet_b2]).reshape(1, -1)
    z64_1 = jnp.zeros((64, 1), _f32)
    w3m = jnp.concatenate([
        jnp.concatenate([enet_W3, z64_1], axis=1),
        jnp.concatenate([z64_1, edgenet_W3], axis=1),
    ], axis=0).astype(_BF)
    weights = (
        w1m, enet_b1.reshape(1, -1), edgenet_b1.reshape(1, -1),
        w2m, b2m, w3m,
        enet_b3.reshape(1, -1), edgenet_b3.reshape(1, -1),
        snet_W1[:F].astype(_BF), snet_W1[F:2 * F].astype(_BF),
        snet_W1[2 * F:], snet_b1.reshape(1, -1),
        snet_W2.astype(_BF), snet_b2.reshape(1, -1),
        snet_W3.astype(_BF), snet_b3.reshape(1, -1),
    )
    weighted, e_x = _tc_mlp(gi, gj, bi, weights)

    wsum_p, dhist = _sc_scatter_pass3(weighted, e_x[:, 0], idx_i)

    return _tc_final(x, wsum_p, dhist, counts)


# speedup vs baseline: 1.2458x; 1.0236x over previous
import functools

import jax
import jax.numpy as jnp
import numpy as np
from jax import lax
from jax.experimental import pallas as pl
from jax.experimental.pallas import tpu as pltpu
from jax.experimental.pallas import tpu_sc as plsc

N = 10000
E = 320000
F = 128

NC = 2
NS = 16
NW = NC * NS
CHUNK = 128
NCHUNKS = E // CHUNK
BASE_CH = NCHUNKS // NW
REM_CH = NCHUNKS - BASE_CH * NW

_SC_MESH = dict(
    mesh=plsc.VectorSubcoreMesh(core_axis_name="c", subcore_axis_name="s"),
    compiler_params=pltpu.CompilerParams(needs_layout_passes=False),
)


NT = 80


def _worker_ids():
    cid = lax.axis_index("c")
    sid = lax.axis_index("s")
    wid = sid * NC + cid
    return cid, sid, wid


def _hist_accum(hist_ref, idx_ref, val16):
    for l in range(CHUNK // 16):
        idx16 = idx_ref[pl.ds(l * 16, 16)]
        plsc.addupdate_scatter(hist_ref, [idx16], val16)


def _k1_body(x_hbm, idxi_hbm, idxj_hbm, zeros2d_hbm, zeros1d_hbm,
             gj_hbm, ssum_hbm, hist_hbm,
             idxi0, idxi1, idxj0, idxj1, rj0, rj1,
             hist_v, ssum_sh, sj0, sj1):
    cid, sid, wid = _worker_ids()
    idxi = (idxi0, idxi1)
    idxj = (idxj0, idxj1)
    rj = (rj0, rj1)
    sj = (sj0, sj1)

    @pl.when(sid == 0)
    def _():
        pltpu.sync_copy(zeros2d_hbm, ssum_sh)

    pltpu.sync_copy(zeros1d_hbm, hist_v)
    plsc.subcore_barrier()

    ones16 = jnp.ones((16,), jnp.float32)

    def load_and_fire(t, b):
        c = wid + NW * t

        @pl.when(c < NCHUNKS)
        def _():
            base = c * CHUNK
            pltpu.sync_copy(idxi_hbm.at[pl.ds(base, CHUNK)], idxi[b])
            pltpu.sync_copy(idxj_hbm.at[pl.ds(base, CHUNK)], idxj[b])
            pltpu.async_copy(x_hbm.at[idxj[b]], rj[b], sj[b])

    for b in range(2):
        load_and_fire(jnp.int32(b), b)

    @pl.loop(0, NT, step=2)
    def _(t0):
        for b in range(2):
            t = t0 + b
            c = wid + NW * t

            @pl.when(c < NCHUNKS)
            def _():
                base = c * CHUNK
                pltpu.make_async_copy(x_hbm.at[idxj[b]], rj[b], sj[b]).wait()
                pltpu.sync_copy(rj[b], gj_hbm.at[pl.ds(base, CHUNK)])
                pltpu.sync_copy(rj[b], ssum_sh.at[idxi[b]], add=True)
                _hist_accum(hist_v, idxi[b], ones16)

            load_and_fire(t + 2, b)

    pltpu.sync_copy(hist_v, hist_hbm.at[wid])
    plsc.subcore_barrier()

    @pl.when(sid == 0)
    def _():
        pltpu.sync_copy(ssum_sh, ssum_hbm.at[cid])


def _k2_body(x_hbm, b_hbm, idxi_hbm, gi_hbm, bi_hbm,
             idxi0, idxi1, rx0, rx1, rb0, rb1, sx0, sx1, sb0, sb1):
    _, _, wid = _worker_ids()
    idxi = (idxi0, idxi1)
    rx = (rx0, rx1)
    rb = (rb0, rb1)
    sx = (sx0, sx1)
    sb = (sb0, sb1)

    def load_and_fire(t, b):
        c = wid + NW * t

        @pl.when(c < NCHUNKS)
        def _():
            base = c * CHUNK
            pltpu.sync_copy(idxi_hbm.at[pl.ds(base, CHUNK)], idxi[b])
            pltpu.async_copy(x_hbm.at[idxi[b]], rx[b], sx[b])
            pltpu.async_copy(b_hbm.at[idxi[b]], rb[b], sb[b])

    for b in range(2):
        load_and_fire(jnp.int32(b), b)

    @pl.loop(0, NT, step=2)
    def _(t0):
        for b in range(2):
            t = t0 + b
            c = wid + NW * t

            @pl.when(c < NCHUNKS)
            def _():
                base = c * CHUNK
                pltpu.make_async_copy(x_hbm.at[idxi[b]], rx[b], sx[b]).wait()
                pltpu.make_async_copy(b_hbm.at[idxi[b]], rb[b], sb[b]).wait()
                pltpu.sync_copy(rx[b], gi_hbm.at[pl.ds(base, CHUNK)])
                pltpu.sync_copy(rb[b], bi_hbm.at[pl.ds(base, CHUNK)])

            load_and_fire(t + 2, b)


def _k3_body(w_hbm, ex_hbm, idxi_hbm, zeros2d_hbm, zeros1d_hbm,
             wsum_hbm, dhist_hbm,
             idxi0, idxi1, r0, r1, ex0, ex1, dhist_v, wsum_sh, s0, s1):
    cid, sid, wid = _worker_ids()
    idxi = (idxi0, idxi1)
    rows = (r0, r1)
    exv = (ex0, ex1)
    sem = (s0, s1)

    @pl.when(sid == 0)
    def _():
        pltpu.sync_copy(zeros2d_hbm, wsum_sh)

    pltpu.sync_copy(zeros1d_hbm, dhist_v)
    plsc.subcore_barrier()

    def load_and_fire(t, b):
        c = wid + NW * t

        @pl.when(c < NCHUNKS)
        def _():
            base = c * CHUNK
            pltpu.sync_copy(idxi_hbm.at[pl.ds(base, CHUNK)], idxi[b])
            pltpu.sync_copy(ex_hbm.at[pl.ds(base, CHUNK)], exv[b])
            pltpu.async_copy(w_hbm.at[pl.ds(base, CHUNK)], rows[b], sem[b])

    for b in range(2):
        load_and_fire(jnp.int32(b), b)

    @pl.loop(0, NT, step=2)
    def _(t0):
        for b in range(2):
            t = t0 + b
            c = wid + NW * t

            @pl.when(c < NCHUNKS)
            def _():
                pltpu.make_async_copy(
                    w_hbm.at[pl.ds(c * CHUNK, CHUNK)], rows[b], sem[b]).wait()
                pltpu.sync_copy(rows[b], wsum_sh.at[idxi[b]], add=True)
                for l in range(CHUNK // 16):
                    idx16 = idxi[b][pl.ds(l * 16, 16)]
                    ex16 = exv[b][pl.ds(l * 16, 16)]
                    plsc.addupdate_scatter(dhist_v, [idx16], ex16)

            load_and_fire(t + 2, b)

    pltpu.sync_copy(dhist_v, dhist_hbm.at[wid])
    plsc.subcore_barrier()

    @pl.when(sid == 0)
    def _():
        pltpu.sync_copy(wsum_sh, wsum_hbm.at[cid])


_f32 = jnp.float32


def _sc_gather_scatter_pass1(x, idx_i, idx_j):
    zeros2d = jnp.zeros((N, F), _f32)
    zeros1d = jnp.zeros((N,), _f32)
    k1 = pl.kernel(
        _k1_body,
        out_type=(
            jax.ShapeDtypeStruct((E, F), _f32),
            jax.ShapeDtypeStruct((NC, N, F), _f32),
            jax.ShapeDtypeStruct((NW, N), _f32),
        ),
        scratch_types=(
            [pltpu.VMEM((CHUNK,), jnp.int32)] * 4
            + [pltpu.VMEM((CHUNK, F), _f32)] * 2
            + [pltpu.VMEM((N,), _f32), pltpu.VMEM_SHARED((N, F), _f32)]
            + [pltpu.SemaphoreType.DMA] * 2
        ),
        **_SC_MESH,
    )
    return k1(x, idx_i, idx_j, zeros2d, zeros1d)


def _sc_gather_xi_b(x, b_tab, idx_i):
    k2 = pl.kernel(
        _k2_body,
        out_type=(
            jax.ShapeDtypeStruct((E, F), _f32),
            jax.ShapeDtypeStruct((E, F), _f32),
        ),
        scratch_types=(
            [pltpu.VMEM((CHUNK,), jnp.int32)] * 2
            + [pltpu.VMEM((CHUNK, F), _f32)] * 4
            + [pltpu.SemaphoreType.DMA] * 4
        ),
        **_SC_MESH,
    )
    return k2(x, b_tab, idx_i)


def _sc_scatter_pass3(weighted, e_x, idx_i):
    zeros2d = jnp.zeros((N, F), _f32)
    zeros1d = jnp.zeros((N,), _f32)
    k3 = pl.kernel(
        _k3_body,
        out_type=(
            jax.ShapeDtypeStruct((NC, N, F), _f32),
            jax.ShapeDtypeStruct((NW, N), _f32),
        ),
        scratch_types=(
            [pltpu.VMEM((CHUNK,), jnp.int32)] * 2
            + [pltpu.VMEM((CHUNK, F), _f32)] * 2
            + [pltpu.VMEM((CHUNK,), _f32)] * 2
            + [pltpu.VMEM((N,), _f32), pltpu.VMEM_SHARED((N, F), _f32)]
            + [pltpu.SemaphoreType.DMA] * 2
        ),
        **_SC_MESH,
    )
    return k3(weighted, e_x, idx_i, zeros2d, zeros1d)


_TROWS = 1000


def _t1_body(x_ref, ssum_ref, histt_ref, w1b_ref, b_ref, counts_ref):
    counts = jnp.sum(histt_ref[...], axis=1, keepdims=True)
    cmax = jnp.maximum(counts, 1.0)
    ga = (counts * x_ref[...] + ssum_ref[0] + ssum_ref[1]) * 0.5 / cmax
    b_ref[...] = jnp.dot(ga, w1b_ref[...], preferred_element_type=_f32)
    counts_ref[...] = counts


def _tc_combine(x, ssum_p, hist, w1b):
    histt = hist.T
    grid = (N // _TROWS,)
    return pl.pallas_call(
        _t1_body,
        grid=grid,
        in_specs=[
            pl.BlockSpec((_TROWS, F), lambda b: (b, 0)),
            pl.BlockSpec((NC, _TROWS, F), lambda b: (0, b, 0)),
            pl.BlockSpec((_TROWS, NW), lambda b: (b, 0)),
            pl.BlockSpec((F, F), lambda b: (0, 0)),
        ],
        out_specs=[
            pl.BlockSpec((_TROWS, F), lambda b: (b, 0)),
            pl.BlockSpec((_TROWS, 1), lambda b: (b, 0)),
        ],
        out_shape=[
            jax.ShapeDtypeStruct((N, F), _f32),
            jax.ShapeDtypeStruct((N, 1), _f32),
        ],
    )(x, ssum_p, histt, w1b)


_EBLK = 8000
_BF = jnp.bfloat16


def _t2_body(gi_ref, gj_ref, bi_ref,
             w1m_ref, eb1_ref, gb1_ref, w2m_ref, b2m_ref, w3m_ref,
             eb3_ref, gb3_ref,
             sw1a_ref, sw1b_ref, sw1c_ref, sb1_ref, sw2_ref, sb2_ref,
             sw3_ref, sb3_ref,
             w_ref, ex_ref):
    gi = gi_ref[...]
    gj = gj_ref[...]
    gi16 = gi.astype(_BF)
    gj16 = gj.astype(_BF)
    nm16 = ((gi + gj) * 0.5).astype(_BF)

    def mm(a, b):
        return jnp.dot(a, b, preferred_element_type=_f32)

    t = mm(nm16, w1m_ref[...])
    h = jnp.maximum(t[:, :128] + eb1_ref[...], 0.0)
    g = jnp.maximum(t[:, 128:] + bi_ref[...] + gb1_ref[...], 0.0)
    hg = jnp.concatenate([h, g], axis=1).astype(_BF)
    u = jnp.maximum(mm(hg, w2m_ref[...]) + b2m_ref[...], 0.0)
    v = mm(u.astype(_BF), w3m_ref[...])
    eij = v[:, 0:1] + eb3_ref[...]
    edge = v[:, 1:2] + gb3_ref[...]
    e_x = jnp.exp(eij)

    s = jnp.maximum(mm(gi16, sw1a_ref[...]) + mm(gj16, sw1b_ref[...])
                    + edge * sw1c_ref[...] + sb1_ref[...], 0.0)
    s = jnp.maximum(mm(s.astype(_BF), sw2_ref[...]) + sb2_ref[...], 0.0)
    m = mm(s.astype(_BF), sw3_ref[...]) + sb3_ref[...]

    w_ref[...] = e_x * m
    ex_ref[...] = e_x


def _tc_mlp(gi, gj, bi, weights):
    grid = (E // _EBLK,)
    eblk = lambda b: (b, 0)
    full = lambda b: (0, 0)
    wspecs = [pl.BlockSpec(w.shape, full) for w in weights]
    return pl.pallas_call(
        _t2_body,
        grid=grid,
        in_specs=[pl.BlockSpec((_EBLK, F), eblk)] * 3 + wspecs,
        out_specs=[
            pl.BlockSpec((_EBLK, F), eblk),
            pl.BlockSpec((_EBLK, 1), eblk),
        ],
        out_shape=[
            jax.ShapeDtypeStruct((E, F), _f32),
            jax.ShapeDtypeStruct((E, 1), _f32),
        ],
    )(gi, gj, bi, *weights)


def _t3_body(x_ref, wsum_ref, dhistt_ref, counts_ref, out_ref):
    dsum = jnp.sum(dhistt_ref[...], axis=1, keepdims=True)
    cmax = jnp.maximum(counts_ref[...], 1.0)
    agg = (wsum_ref[0] + wsum_ref[1]) / ((dsum + 1e-9) * cmax)
    out_ref[...] = jnp.maximum(x_ref[...] + agg, 0.0)


def _tc_final(x, wsum_p, dhist, counts):
    dhistt = dhist.T
    grid = (N // _TROWS,)
    return pl.pallas_call(
        _t3_body,
        grid=grid,
        in_specs=[
            pl.BlockSpec((_TROWS, F), lambda b: (b, 0)),
            pl.BlockSpec((NC, _TROWS, F), lambda b: (0, b, 0)),
            pl.BlockSpec((_TROWS, NW), lambda b: (b, 0)),
            pl.BlockSpec((_TROWS, 1), lambda b: (b, 0)),
        ],
        out_specs=pl.BlockSpec((_TROWS, F), lambda b: (b, 0)),
        out_shape=jax.ShapeDtypeStruct((N, F), _f32),
    )(x, wsum_p, dhistt, counts)


def kernel(x, edge_index, e,
           enet_W1, enet_b1, enet_W2, enet_b2, enet_W3, enet_b3,
           edgenet_W1, edgenet_b1, edgenet_W2, edgenet_b2, edgenet_W3, edgenet_b3,
           snet_W1, snet_b1, snet_W2, snet_b2, snet_W3, snet_b3):
    idx_i = edge_index[0]
    idx_j = edge_index[1]

    gj, ssum_p, hist = _sc_gather_scatter_pass1(x, idx_i, idx_j)

    b_tab, counts = _tc_combine(x, ssum_p, hist, edgenet_W1[F:])

    gi, bi = _sc_gather_xi_b(x, b_tab, idx_i)

    w1m = jnp.concatenate([enet_W1, edgenet_W1[:F]], axis=1).astype(_BF)
    z128_64 = jnp.zeros((F, 64), _f32)
    w2m = jnp.concatenate([
        jnp.concatenate([enet_W2, z128_64], axis=1),
        jnp.concatenate([z128_64, edgenet_W2], axis=1),
    ], axis=0).astype(_BF)
    b2m = jnp.concatenate([enet_b2, edgenet_b2]).reshape(1, -1)
    z64_1 = jnp.zeros((64, 1), _f32)
    w3m = jnp.concatenate([
        jnp.concatenate([enet_W3, z64_1], axis=1),
        jnp.concatenate([z64_1, edgenet_W3], axis=1),
    ], axis=0).astype(_BF)
    weights = (
        w1m, enet_b1.reshape(1, -1), edgenet_b1.reshape(1, -1),
        w2m, b2m, w3m,
        enet_b3.reshape(1, -1), edgenet_b3.reshape(1, -1),
        snet_W1[:F].astype(_BF), snet_W1[F:2 * F].astype(_BF),
        snet_W1[2 * F:], snet_b1.reshape(1, -1),
        snet_W2.astype(_BF), snet_b2.reshape(1, -1),
        snet_W3.astype(_BF), snet_b3.reshape(1, -1),
    )
    weighted, e_x = _tc_mlp(gi, gj, bi, weights)

    wsum_p, dhist = _sc_scatter_pass3(weighted, e_x[:, 0], idx_i)

    return _tc_final(x, wsum_p, dhist, counts)
